# Initial kernel scaffold; baseline (speedup 1.0000x reference)
import jax
import jax.numpy as jnp
from jax.experimental import pallas as pl


def kernel(x_o, x_c, edge_index_o, edge_index_c, batch_o, W1o, b1o, W2o, b2o, W1c, b1c, W2c, b2c, Wl, bl):
    raise NotImplementedError("write your pallas kernel here")


# speedup vs baseline: 43.7229x; 43.7229x over previous
import functools

import jax
import jax.numpy as jnp
from jax import lax
from jax.experimental import pallas as pl
from jax.experimental.pallas import tpu as pltpu
from jax.experimental.pallas import tpu_sc as plsc

N = 10000
E = 320000
DF = 128
H = 256
C = 10
G = 64

NT = 32
CH = 80
EPT = CH * 128
PADE = NT * EPT - E
NPAD = 10240
DEGSZ = 10240
AROWS = NPAD // 16
PSZ = G * NPAD
PROWS = PSZ // 16

@functools.cache
def _mesh():
    return plsc.VectorSubcoreMesh(core_axis_name="c", subcore_axis_name="s",
                                  num_cores=2, num_subcores=16)


def _drain(sem, src_hbm, dst_vmem):
    pltpu.make_async_copy(src_hbm, dst_vmem, sem).wait()


@functools.cache
def _sc_deg_built():
    return pl.kernel(
        _sc_deg,
        out_type=jax.ShapeDtypeStruct((2, DEGSZ), jnp.float32),
        mesh=_mesh(),
        compiler_params=pltpu.CompilerParams(needs_layout_passes=False),
        scratch_types=[
            pltpu.VMEM((CH, 128), jnp.int32),
            pltpu.VMEM((1, 128), jnp.float32),
            pltpu.VMEM_SHARED((DEGSZ,), jnp.float32),
            pltpu.SemaphoreType.DMA,
        ],
    )


def _sc_deg(dstp_hbm, ones_hbm, zd_hbm, deg_out, dst_v, ones_v, deg_sh, sem):
    c = lax.axis_index("c")
    s = lax.axis_index("s")
    wid = s * 2 + c
    pltpu.sync_copy(zd_hbm, deg_sh.at[pl.ds(s * 640, 640)])
    pltpu.sync_copy(dstp_hbm.at[wid], dst_v)
    pltpu.sync_copy(ones_hbm, ones_v)
    plsc.subcore_barrier()

    def issue(j):
        pltpu.async_copy(ones_v.at[0], deg_sh.at[dst_v.at[j]], sem, add=True)

    for j in range(8):
        issue(j)

    def body(j, carry):
        _drain(sem, ones_hbm.at[0], ones_v.at[0])
        issue(j)
        return carry

    lax.fori_loop(8, CH, body, 0)
    for _ in range(8):
        _drain(sem, ones_hbm.at[0], ones_v.at[0])
    plsc.subcore_barrier()
    pltpu.sync_copy(deg_sh.at[pl.ds(s * 640, 640)],
                    deg_out.at[c, pl.ds(s * 640, 640)])


@functools.cache
def _sc_agg_built():
    return pl.kernel(
        _sc_agg,
        out_type=jax.ShapeDtypeStruct((2, NPAD, DF), jnp.float32),
        mesh=_mesh(),
        compiler_params=pltpu.CompilerParams(needs_layout_passes=False),
        scratch_types=[
            pltpu.VMEM((CH, 128), jnp.int32),
            pltpu.VMEM((CH, 128), jnp.int32),
            pltpu.VMEM((128, DF), jnp.float32),
            pltpu.VMEM_SHARED((NPAD, DF), jnp.float32),
            pltpu.SemaphoreType.DMA,
            pltpu.SemaphoreType.DMA,
        ],
    )


def _sc_agg(srcp_hbm, dstp_hbm, xs_hbm, za_hbm, agg_out,
            src_v, dst_v, stage, agg_sh, semg, sems):
    c = lax.axis_index("c")
    s = lax.axis_index("s")
    wid = s * 2 + c
    pltpu.sync_copy(srcp_hbm.at[wid], src_v)
    pltpu.sync_copy(dstp_hbm.at[wid], dst_v)
    pltpu.sync_copy(za_hbm, agg_sh.at[pl.ds(s * AROWS, AROWS)])
    plsc.subcore_barrier()

    def body(j, carry):
        pltpu.async_copy(xs_hbm.at[src_v.at[j]], stage, semg)
        _drain(semg, xs_hbm.at[pl.ds(0, 128)], stage)
        pltpu.async_copy(stage, agg_sh.at[dst_v.at[j]], sems, add=True)
        _drain(sems, xs_hbm.at[pl.ds(0, 128)], stage)
        return carry

    lax.fori_loop(0, CH, body, 0)
    plsc.subcore_barrier()
    pltpu.sync_copy(agg_sh.at[pl.ds(s * AROWS, AROWS)],
                    agg_out.at[c, pl.ds(s * AROWS, AROWS)])


@functools.cache
def _sc_p_built():
    return pl.kernel(
        _sc_p,
        out_type=jax.ShapeDtypeStruct((2, PSZ), jnp.float32),
        mesh=_mesh(),
        compiler_params=pltpu.CompilerParams(needs_layout_passes=False),
        scratch_types=[
            pltpu.VMEM((CH, 128), jnp.int32),
            pltpu.VMEM((CH, 128), jnp.int32),
            pltpu.VMEM((NPAD,), jnp.float32),
            pltpu.VMEM((NPAD,), jnp.int32),
            pltpu.VMEM((CH, 128), jnp.float32),
            pltpu.VMEM((CH, 128), jnp.int32),
            pltpu.VMEM_SHARED((PSZ,), jnp.float32),
            pltpu.SemaphoreType.DMA,
        ],
    )


def _sc_p(srcp_hbm, dstp_hbm, dinv_hbm, batch_hbm, zp_hbm, p_out,
          src_v, dst_v, dinv_v, batch_v, wbuf, pibuf, p_sh, semp):
    c = lax.axis_index("c")
    s = lax.axis_index("s")
    wid = s * 2 + c
    pltpu.sync_copy(srcp_hbm.at[wid], src_v)
    pltpu.sync_copy(dstp_hbm.at[wid], dst_v)
    pltpu.sync_copy(dinv_hbm, dinv_v)
    pltpu.sync_copy(batch_hbm, batch_v)
    pltpu.sync_copy(zp_hbm, p_sh.at[pl.ds(s * PROWS, PROWS)])
    plsc.subcore_barrier()

    def chunk(j, drain_p):
        for k in range(8):
            sl = pl.ds(k * 16, 16)
            s16 = src_v[j, sl]
            d16 = dst_v[j, sl]
            fs = plsc.load_gather(dinv_v, [s16])
            fd = plsc.load_gather(dinv_v, [d16])
            g16 = plsc.load_gather(batch_v, [d16])
            wbuf[j, sl] = fs * fd
            pibuf[j, sl] = g16 * NPAD + s16
        pltpu.async_copy(wbuf.at[j], p_sh.at[pibuf.at[j]], semp, add=True)
        if drain_p:
            _drain(semp, dinv_hbm.at[pl.ds(0, 128)], wbuf.at[0])

    def body(j, carry, drain_p):
        chunk(j, drain_p)
        return carry

    lax.fori_loop(0, 16, functools.partial(body, drain_p=False), 0)
    lax.fori_loop(16, CH, functools.partial(body, drain_p=True), 0)
    for _ in range(16):
        _drain(semp, dinv_hbm.at[pl.ds(0, 128)], wbuf.at[0])
    plsc.subcore_barrier()
    pltpu.sync_copy(p_sh.at[pl.ds(s * PROWS, PROWS)],
                    p_out.at[c, pl.ds(s * PROWS, PROWS)])


def _tc_scale_body(deg_ref, x_ref, dinv_ref, xs_ref):
    d = deg_ref[:, 0:1] + deg_ref[:, 1:2] + 1.0
    rows = lax.broadcasted_iota(jnp.int32, (DEGSZ, 1), 0)
    dv = jnp.where(rows < N, lax.rsqrt(d), 0.0)
    dinv_ref[...] = dv
    xs_ref[...] = x_ref[...] * dv[:N]


def _tc_scale(deg_colT, x_c):
    return pl.pallas_call(
        _tc_scale_body,
        out_shape=[
            jax.ShapeDtypeStruct((NPAD, 1), jnp.float32),
            jax.ShapeDtypeStruct((N, DF), jnp.float32),
        ],
    )(deg_colT, x_c)


BK = 2048


def _tc_final_body(x_ref, agg_ref, dvc_ref, dvr_ref, b_ref, p_ref,
                   w1_ref, b1_ref, w2_ref, b2_ref, wl_ref, bl_ref,
                   o_ref, acc, cnt):
    i = pl.program_id(0)

    @pl.when(i == 0)
    def _init():
        acc[...] = jnp.zeros_like(acc)
        cnt[...] = jnp.zeros_like(cnt)

    dv = dvc_ref[...]
    z = dv * (agg_ref[0] + agg_ref[1]) + (dv * dv) * x_ref[...]
    h1 = jnp.maximum(
        jnp.dot(z, w1_ref[...], preferred_element_type=jnp.float32)
        + b1_ref[...], 0.0)
    bb = b_ref[...]
    gi = lax.broadcasted_iota(jnp.int32, (G, BK), 0)
    mf = (gi == bb).astype(jnp.float32)
    dvr = dvr_ref[...]
    pc = p_ref[0] + p_ref[1] + mf * (dvr * dvr)
    acc[...] += jnp.dot(pc, h1, preferred_element_type=jnp.float32)
    cnt[...] += jnp.sum(mf, axis=1, keepdims=True)

    @pl.when(i == NPAD // BK - 1)
    def _fin():
        pooled = (jnp.dot(acc[...], w2_ref[...],
                          preferred_element_type=jnp.float32)
                  + jnp.dot(cnt[...], b2_ref[...],
                            preferred_element_type=jnp.float32))
        o_ref[...] = (jnp.dot(pooled, wl_ref[...],
                              preferred_element_type=jnp.float32)
                      + bl_ref[...])


def _tc_final(x_c, agg2, dinv_col, dinv_row, batch_row, p3,
              W1, b1, W2, b2, Wl, bl):
    nblk = NPAD // BK
    return pl.pallas_call(
        _tc_final_body,
        grid=(nblk,),
        in_specs=[
            pl.BlockSpec((BK, DF), lambda i: (i, 0)),
            pl.BlockSpec((2, BK, DF), lambda i: (0, i, 0)),
            pl.BlockSpec((BK, 1), lambda i: (i, 0)),
            pl.BlockSpec((1, BK), lambda i: (0, i)),
            pl.BlockSpec((1, BK), lambda i: (0, i)),
            pl.BlockSpec((2, G, BK), lambda i: (0, 0, i)),
            pl.BlockSpec((DF, H), lambda i: (0, 0)),
            pl.BlockSpec((1, H), lambda i: (0, 0)),
            pl.BlockSpec((H, H), lambda i: (0, 0)),
            pl.BlockSpec((1, H), lambda i: (0, 0)),
            pl.BlockSpec((H, C), lambda i: (0, 0)),
            pl.BlockSpec((1, C), lambda i: (0, 0)),
        ],
        out_specs=pl.BlockSpec((G, C), lambda i: (0, 0)),
        out_shape=jax.ShapeDtypeStruct((G, C), jnp.float32),
        scratch_shapes=[
            pltpu.VMEM((G, H), jnp.float32),
            pltpu.VMEM((G, 1), jnp.float32),
        ],
    )(x_c, agg2, dinv_col, dinv_row, batch_row, p3,
      W1, b1, W2, b2, Wl, bl)


def kernel(x_o, x_c, edge_index_o, edge_index_c, batch_o,
           W1o, b1o, W2o, b2o, W1c, b1c, W2c, b2c, Wl, bl):
    src = edge_index_c[0]
    dst = edge_index_c[1]
    ar = jnp.arange(PADE, dtype=jnp.int32)
    srcp = jnp.concatenate([src, (ar * 13) % N]).reshape(NT, CH, 128)
    dstp = jnp.concatenate([dst, N + (ar % 128)]).reshape(NT, CH, 128)
    ones_row = jnp.ones((1, 128), jnp.float32)
    zd = jnp.zeros((640,), jnp.float32)

    deg_parts = _sc_deg_built()(dstp, ones_row, zd)
    dinv_col, xs = _tc_scale(deg_parts.T, x_c)

    dinv_flat = dinv_col.reshape(NPAD)
    batch_flat = jnp.pad(batch_o, (0, NPAD - N))
    za = jnp.zeros((AROWS, DF), jnp.float32)
    zp = jnp.zeros((PROWS,), jnp.float32)
    x_pad = jnp.pad(x_c, ((0, NPAD - N), (0, 0)))
    agg_parts = _sc_agg_built()(srcp, dstp, xs, za)
    p_parts = _sc_p_built()(srcp, dstp, dinv_flat, batch_flat, zp)

    p3 = p_parts.reshape(2, G, NPAD)
    dinv_row = dinv_col.reshape(1, NPAD)
    batch_row = jnp.pad(batch_o, (0, NPAD - N),
                        constant_values=-1).reshape(1, NPAD)
    return _tc_final(x_pad, agg_parts, dinv_col, dinv_row, batch_row, p3,
                     W1c, b1c.reshape(1, H), W2c, b2c.reshape(1, H),
                     Wl, bl.reshape(1, C))
